# Initial kernel scaffold; baseline (speedup 1.0000x reference)
import jax
import jax.numpy as jnp
from jax.experimental import pallas as pl


def kernel(input_tensor, emb_table, W1, b1, W2, b2):
    raise NotImplementedError("write your pallas kernel here")


# speedup vs baseline: 2.4115x; 2.4115x over previous
import functools

import jax
import jax.numpy as jnp
from jax import lax
from jax.experimental import pallas as pl
from jax.experimental.pallas import tpu as pltpu
from jax.experimental.pallas import tpu_sc as plsc

VOCAB = 1000000
D = 128
B = 16384
NC = 2
NS = 16
NW = NC * NS
B_PER_W = B // NW
CHUNK = 256
N_CHUNKS = B_PER_W // CHUNK


def _sc_gather_sum(emb_table, idx0, idx1):
  mesh = plsc.VectorSubcoreMesh(core_axis_name="c", subcore_axis_name="s")

  @functools.partial(
      pl.kernel,
      out_type=jax.ShapeDtypeStruct((B, D), jnp.float32),
      mesh=mesh,
      scratch_types=[
          pltpu.VMEM((CHUNK,), jnp.int32),
          pltpu.VMEM((CHUNK,), jnp.int32),
          pltpu.VMEM((CHUNK, D), jnp.float32),
          pltpu.VMEM((CHUNK, D), jnp.float32),
          pltpu.SemaphoreType.DMA,
          pltpu.SemaphoreType.DMA,
      ],
  )
  def k(table_hbm, idx0_hbm, idx1_hbm, out_hbm, idx0_v, idx1_v, buf0, buf1,
        sem0, sem1):
    wid = lax.axis_index("s") * NC + lax.axis_index("c")
    base_w = wid * B_PER_W

    def chunk_body(g, carry):
      base = base_w + g * CHUNK
      pltpu.sync_copy(idx0_hbm.at[pl.ds(base, CHUNK)], idx0_v)
      pltpu.sync_copy(idx1_hbm.at[pl.ds(base, CHUNK)], idx1_v)
      cp0 = pltpu.async_copy(table_hbm.at[idx0_v], buf0, sem0)
      cp1 = pltpu.async_copy(table_hbm.at[idx1_v], buf1, sem1)
      cp0.wait()
      cp1.wait()

      def add_body(s, c2):
        for c in range(D // 16):
          sl = pl.ds(c * 16, 16)
          buf0[s, sl] = buf0[s, sl] + buf1[s, sl]
        return c2

      lax.fori_loop(0, CHUNK, add_body, 0, unroll=False)
      pltpu.sync_copy(buf0, out_hbm.at[pl.ds(base, CHUNK)])
      return carry

    lax.fori_loop(0, N_CHUNKS, chunk_body, 0, unroll=False)

  return k(emb_table, idx0, idx1)


BLK = 2048


def _mlp_body(s_ref, w1_ref, b1_ref, w2_ref, b2_ref, o_ref):
  s = s_ref[...]
  h = jnp.dot(s, w1_ref[...], preferred_element_type=jnp.float32) * 0.5
  h = jnp.maximum(h + b1_ref[...], 0.0)
  z = jnp.sum(h * w2_ref[...], axis=1, keepdims=True) + b2_ref[...]
  o_ref[...] = jnp.maximum(jax.nn.sigmoid(z), 0.0)


def _tc_mlp(s, W1, b1, W2, b2):
  grid = (B // BLK,)
  return pl.pallas_call(
      _mlp_body,
      grid=grid,
      in_specs=[
          pl.BlockSpec((BLK, D), lambda i: (i, 0)),
          pl.BlockSpec((D, 16), lambda i: (0, 0)),
          pl.BlockSpec((1, 16), lambda i: (0, 0)),
          pl.BlockSpec((1, 16), lambda i: (0, 0)),
          pl.BlockSpec((1, 1), lambda i: (0, 0)),
      ],
      out_specs=pl.BlockSpec((BLK, 1), lambda i: (i, 0)),
      out_shape=jax.ShapeDtypeStruct((B, 1), jnp.float32),
  )(s, W1, b1, W2, b2)


def kernel(input_tensor, emb_table, W1, b1, W2, b2):
  idx = input_tensor.astype(jnp.int32)
  idx0 = idx[:, 0]
  idx1 = idx[:, 1]
  s = _sc_gather_sum(emb_table, idx0, idx1)
  return _tc_mlp(s, W1, b1.reshape(1, 16), W2.reshape(1, 16),
                 b2.reshape(1, 1))
